# baseline (speedup 1.0000x reference)
import functools

import jax
import jax.numpy as jnp
from jax import lax
from jax.experimental import pallas as pl
from jax.experimental.pallas import tpu as pltpu
from jax.experimental.pallas import tpu_sc as plsc

_B = 16384
_D = 64
_DP = 128
_NT = 10
_NC = 2
_NS = 16
_NW = _NC * _NS
_CHUNK = 128
_NBUF = 4

_GW = _NT * _D
_DW = 32
_H1 = 192
_H2 = 256

_NSPLIT = 2
_S = _B // _NSPLIT


def _sc_gather_body(rpw, thbm, idx_hbm, out_hbm, idx_all, a0, a1, b0, b1,
                    ga0, ga1, gb0, gb1, ws0, ws1, isem):
    abufs = (a0, a1)
    bbufs = (b0, b1)
    gasems = (ga0, ga1)
    gbsems = (gb0, gb1)
    wsems = (ws0, ws1)
    nchunk = rpw // _CHUNK
    wid = lax.axis_index("s") * _NC + lax.axis_index("c")
    wbase = wid * rpw
    pltpu.async_copy(idx_hbm.at[:, pl.ds(wbase, rpw)], idx_all, isem).wait()

    steps = [(p, c) for p in range(_NT // 2) for c in range(nchunk)]
    gacp = [None] * len(steps)
    gbcp = [None] * len(steps)
    wcp = [None] * len(steps)

    def fire_gathers(s):
        p, c = steps[s]
        k = s % 2
        gacp[s] = pltpu.async_copy(
            thbm.at[idx_all.at[2 * p, pl.ds(c * _CHUNK, _CHUNK)]],
            abufs[k], gasems[k])
        gbcp[s] = pltpu.async_copy(
            thbm.at[idx_all.at[2 * p + 1, pl.ds(c * _CHUNK, _CHUNK)]],
            bbufs[k], gbsems[k])

    def merge_write(s):
        p, c = steps[s]
        k = s % 2
        a, b = abufs[k], bbufs[k]
        gacp[s].wait()
        gbcp[s].wait()

        def mrow(r, carry):
            for c4 in range(_D // 16):
                a[r, pl.ds(_D + 16 * c4, 16)] = b[r, pl.ds(16 * c4, 16)]
            return carry
        lax.fori_loop(0, _CHUNK, mrow, 0)
        wcp[s] = pltpu.async_copy(
            a,
            out_hbm.at[pl.ds(wbase + c * _CHUNK, _CHUNK),
                       pl.ds(p * 2 * _D, 2 * _D)],
            wsems[k])

    for s in range(len(steps)):
        if s >= 2:
            wcp[s - 2].wait()
        fire_gathers(s)
        if s >= 1:
            merge_write(s - 1)
    merge_write(len(steps) - 1)
    for s in range(max(0, len(steps) - 2), len(steps)):
        wcp[s].wait()


@functools.cache
def _get_sc_gather(rows):
    rpw = rows // _NW
    return pl.kernel(
        functools.partial(_sc_gather_body, rpw),
        mesh=plsc.VectorSubcoreMesh(core_axis_name="c", subcore_axis_name="s"),
        out_type=jax.ShapeDtypeStruct((rows, _GW), jnp.float32),
        scratch_types=[
            pltpu.VMEM((_NT, rpw), jnp.int32),
            pltpu.VMEM((_CHUNK, _DP), jnp.float32),
            pltpu.VMEM((_CHUNK, _DP), jnp.float32),
            pltpu.VMEM((_CHUNK, _DP), jnp.float32),
            pltpu.VMEM((_CHUNK, _DP), jnp.float32),
            pltpu.SemaphoreType.DMA,
            pltpu.SemaphoreType.DMA,
            pltpu.SemaphoreType.DMA,
            pltpu.SemaphoreType.DMA,
            pltpu.SemaphoreType.DMA,
            pltpu.SemaphoreType.DMA,
            pltpu.SemaphoreType.DMA,
        ],
    )


_BLK = 1024


def _mlp_body(g_ref, d_ref, wg_ref, wd_ref, bd_ref, whd_ref, we_ref,
              bh_ref, wo_ref, bo_ref, o_ref):
    d = d_ref[...]
    a = jnp.maximum(
        jnp.dot(d, wd_ref[...], preferred_element_type=jnp.float32)
        + bd_ref[...], 0.0)
    h = jnp.dot(g_ref[...], wg_ref[...], preferred_element_type=jnp.float32)
    h = h + jnp.dot(a, whd_ref[...], preferred_element_type=jnp.float32)
    h = h + jnp.dot(d, we_ref[...], preferred_element_type=jnp.float32)
    h = jnp.maximum(h + bh_ref[...], 0.0)
    o = jnp.dot(h, wo_ref[...], preferred_element_type=jnp.float32) + bo_ref[...]
    o_ref[...] = jax.nn.sigmoid(o)


@functools.cache
def _get_mlp(rows):
    return pl.pallas_call(
        _mlp_body,
        grid=(rows // _BLK,),
        in_specs=[
            pl.BlockSpec((_BLK, _GW), lambda i: (i, 0)),
            pl.BlockSpec((_BLK, _DW), lambda i: (i, 0)),
            pl.BlockSpec((_GW, _H2), lambda i: (0, 0)),
            pl.BlockSpec((_DW, _H1), lambda i: (0, 0)),
            pl.BlockSpec((1, _H1), lambda i: (0, 0)),
            pl.BlockSpec((_H1, _H2), lambda i: (0, 0)),
            pl.BlockSpec((_DW, _H2), lambda i: (0, 0)),
            pl.BlockSpec((1, _H2), lambda i: (0, 0)),
            pl.BlockSpec((_H2, 1), lambda i: (0, 0)),
            pl.BlockSpec((1, 1), lambda i: (0, 0)),
        ],
        out_specs=pl.BlockSpec((_BLK, 1), lambda i: (i, 0)),
        out_shape=jax.ShapeDtypeStruct((rows, 1), jnp.float32),
    )


def kernel(user_id, amount, merchant_id, merchant_city, merchant_state, mcc,
           zip_2, zip_4, user_avg_amount, merchant_avg_amount, card_id,
           use_chip, zip_1, errors, E_user_id, E_amount, E_mer_id, E_mer_ct,
           E_mer_st, E_mcc, E_zip2, E_zip4, E_user_avg, E_mer_avg, W_card,
           b_card, W_chip, b_chip, W_zip1, b_zip1, W_hidden, b_hidden, W_out,
           b_out):
    tables = [E_user_id, E_amount, E_mer_id, E_mer_ct, E_mer_st, E_mcc,
              E_zip2, E_zip4, E_user_avg, E_mer_avg]
    offs, acc = [], 0
    for t in tables:
        offs.append(acc)
        acc += t.shape[0]
    thbm = jnp.pad(jnp.concatenate(tables, axis=0), ((0, 0), (0, _DP - _D)))

    idx = (jnp.concatenate(
        [user_id, amount, merchant_id, merchant_city, merchant_state, mcc,
         zip_2, zip_4, user_avg_amount, merchant_avg_amount],
        axis=1).astype(jnp.int32)
        + jnp.asarray(offs, jnp.int32)[None, :]).T

    dpack = jnp.concatenate(
        [card_id, use_chip, zip_1, errors,
         jnp.zeros((_B, _DW - 17), jnp.float32)], axis=1)

    Wh = W_hidden
    segs = [
        Wh[0:64],
        Wh[128:192],
        Wh[200:264],
        Wh[264:328],
        Wh[328:392],
        Wh[392:456] + Wh[456:520],
        Wh[648:712],
        Wh[712:776],
        Wh[776:840],
        Wh[840:904],
    ]
    wg = jnp.concatenate(segs, axis=0)
    whd = jnp.concatenate([Wh[64:128], Wh[520:584], Wh[584:648]],
                          axis=0)
    we = jnp.zeros((_DW, _H2), jnp.float32).at[9:17].set(Wh[192:200])
    wd = (jnp.zeros((_DW, _H1), jnp.float32)
          .at[0:4, 0:64].set(W_card)
          .at[4:7, 64:128].set(W_chip)
          .at[7:9, 128:192].set(W_zip1))
    bd = jnp.concatenate([b_card, b_chip, b_zip1])[None, :]
    bh = b_hidden[None, :]
    bo = b_out[None, :]

    sc = _get_sc_gather(_S)
    mlp = _get_mlp(_S)
    outs = []
    for i in range(_NSPLIT):
        g = sc(thbm, lax.slice(idx, (0, i * _S), (_NT, (i + 1) * _S)))
        outs.append(mlp(g, lax.slice(dpack, (i * _S, 0), ((i + 1) * _S, _DW)),
                        wg, wd, bd, whd, we, bh, W_out, bo))
    return jnp.concatenate(outs, axis=0)


# speedup vs baseline: 1.5914x; 1.5914x over previous
import functools

import jax
import jax.numpy as jnp
from jax import lax
from jax.experimental import pallas as pl
from jax.experimental.pallas import tpu as pltpu
from jax.experimental.pallas import tpu_sc as plsc

_B = 16384
_D = 64
_DP = 128
_NT = 7
_NC = 2
_NS = 16
_NW = _NC * _NS
_CHUNK = 128

_GW = 4 * _DP
_DW = 32
_H1 = 192
_H2 = 256
_V1 = 128
_V2 = 256

_NSPLIT = 2
_S = _B // _NSPLIT


def _sc_gather_body(rpw, t0, t1, t2, t3, t4, t5, t6,
                    idx_hbm, out_hbm, idx_all, a0, a1, b0, b1,
                    ga0, ga1, gb0, gb1, ws0, ws1, isem):
    tables = (t0, t1, t2, t3, t4, t5, t6)
    abufs = (a0, a1)
    bbufs = (b0, b1)
    gasems = (ga0, ga1)
    gbsems = (gb0, gb1)
    wsems = (ws0, ws1)
    nchunk = rpw // _CHUNK
    wid = lax.axis_index("s") * _NC + lax.axis_index("c")
    wbase = wid * rpw
    pltpu.async_copy(idx_hbm.at[:, pl.ds(wbase, rpw)], idx_all, isem).wait()

    steps = [(p, c) for p in range(4) for c in range(nchunk)]
    gacp = [None] * len(steps)
    gbcp = [None] * len(steps)
    wcp = [None] * len(steps)

    def fire_gathers(s):
        p, c = steps[s]
        k = s % 2
        gacp[s] = pltpu.async_copy(
            tables[2 * p].at[idx_all.at[2 * p, pl.ds(c * _CHUNK, _CHUNK)]],
            abufs[k], gasems[k])
        if p < 3:
            gbcp[s] = pltpu.async_copy(
                tables[2 * p + 1].at[idx_all.at[2 * p + 1,
                                                pl.ds(c * _CHUNK, _CHUNK)]],
                bbufs[k], gbsems[k])

    def merge_write(s):
        p, c = steps[s]
        k = s % 2
        a, b = abufs[k], bbufs[k]
        gacp[s].wait()
        if p < 3:
            gbcp[s].wait()

            def mrow(r, carry):
                for c4 in range(_D // 16):
                    a[r, pl.ds(_D + 16 * c4, 16)] = b[r, pl.ds(16 * c4, 16)]
                return carry
            lax.fori_loop(0, _CHUNK, mrow, 0)
        wcp[s] = pltpu.async_copy(
            a,
            out_hbm.at[pl.ds(wbase + c * _CHUNK, _CHUNK),
                       pl.ds(p * _DP, _DP)],
            wsems[k])

    for s in range(len(steps)):
        if s >= 2:
            wcp[s - 2].wait()
        fire_gathers(s)
        if s >= 1:
            merge_write(s - 1)
    merge_write(len(steps) - 1)
    for s in range(max(0, len(steps) - 2), len(steps)):
        wcp[s].wait()


@functools.cache
def _get_sc_gather(rows):
    rpw = rows // _NW
    return pl.kernel(
        functools.partial(_sc_gather_body, rpw),
        mesh=plsc.VectorSubcoreMesh(core_axis_name="c", subcore_axis_name="s"),
        out_type=jax.ShapeDtypeStruct((rows, _GW), jnp.float32),
        scratch_types=[
            pltpu.VMEM((10, rpw), jnp.int32),
            pltpu.VMEM((_CHUNK, _DP), jnp.float32),
            pltpu.VMEM((_CHUNK, _DP), jnp.float32),
            pltpu.VMEM((_CHUNK, _DP), jnp.float32),
            pltpu.VMEM((_CHUNK, _DP), jnp.float32),
            pltpu.SemaphoreType.DMA,
            pltpu.SemaphoreType.DMA,
            pltpu.SemaphoreType.DMA,
            pltpu.SemaphoreType.DMA,
            pltpu.SemaphoreType.DMA,
            pltpu.SemaphoreType.DMA,
            pltpu.SemaphoreType.DMA,
        ],
    )


_BLK = 1024


def _mlp_body(g_ref, d_ref, imst_ref, imcc_ref, iz4_ref,
              tmst_ref, tmcc_ref, tz4_ref,
              wmst_ref, wmcc_ref, wz4_ref, wg_ref, wd_ref, bd_ref, whd_ref,
              we_ref, bh_ref, wo_ref, bo_ref, o_ref):
    d = d_ref[...]
    a = jnp.maximum(
        jnp.dot(d, wd_ref[...], preferred_element_type=jnp.float32)
        + bd_ref[...], 0.0)
    h = jnp.dot(g_ref[...], wg_ref[...], preferred_element_type=jnp.float32)

    oh_mst = (imst_ref[...]
              == lax.broadcasted_iota(jnp.int32, (_BLK, _V2), 1)
              ).astype(jnp.float32)
    oh_mcc = (imcc_ref[...]
              == lax.broadcasted_iota(jnp.int32, (_BLK, _V1), 1)
              ).astype(jnp.float32)
    oh_z4 = (iz4_ref[...]
             == lax.broadcasted_iota(jnp.int32, (_BLK, _V1), 1)
             ).astype(jnp.float32)
    gmst = jnp.dot(oh_mst, tmst_ref[...], preferred_element_type=jnp.float32)
    gmcc = jnp.dot(oh_mcc, tmcc_ref[...], preferred_element_type=jnp.float32)
    gz4 = jnp.dot(oh_z4, tz4_ref[...], preferred_element_type=jnp.float32)
    h = h + jnp.dot(gmst, wmst_ref[...], preferred_element_type=jnp.float32)
    h = h + jnp.dot(gmcc, wmcc_ref[...], preferred_element_type=jnp.float32)
    h = h + jnp.dot(gz4, wz4_ref[...], preferred_element_type=jnp.float32)

    h = h + jnp.dot(a, whd_ref[...], preferred_element_type=jnp.float32)
    h = h + jnp.dot(d, we_ref[...], preferred_element_type=jnp.float32)
    h = jnp.maximum(h + bh_ref[...], 0.0)
    o = jnp.dot(h, wo_ref[...], preferred_element_type=jnp.float32) + bo_ref[...]
    o_ref[...] = jax.nn.sigmoid(o)


@functools.cache
def _get_mlp(rows):
    return pl.pallas_call(
        _mlp_body,
        grid=(rows // _BLK,),
        in_specs=[
            pl.BlockSpec((_BLK, _GW), lambda i: (i, 0)),
            pl.BlockSpec((_BLK, _DW), lambda i: (i, 0)),
            pl.BlockSpec((_BLK, 1), lambda i: (i, 0)),
            pl.BlockSpec((_BLK, 1), lambda i: (i, 0)),
            pl.BlockSpec((_BLK, 1), lambda i: (i, 0)),
            pl.BlockSpec((_V2, _D), lambda i: (0, 0)),
            pl.BlockSpec((_V1, _D), lambda i: (0, 0)),
            pl.BlockSpec((_V1, _D), lambda i: (0, 0)),
            pl.BlockSpec((_D, _H2), lambda i: (0, 0)),
            pl.BlockSpec((_D, _H2), lambda i: (0, 0)),
            pl.BlockSpec((_D, _H2), lambda i: (0, 0)),
            pl.BlockSpec((_GW, _H2), lambda i: (0, 0)),
            pl.BlockSpec((_DW, _H1), lambda i: (0, 0)),
            pl.BlockSpec((1, _H1), lambda i: (0, 0)),
            pl.BlockSpec((_H1, _H2), lambda i: (0, 0)),
            pl.BlockSpec((_DW, _H2), lambda i: (0, 0)),
            pl.BlockSpec((1, _H2), lambda i: (0, 0)),
            pl.BlockSpec((_H2, 1), lambda i: (0, 0)),
            pl.BlockSpec((1, 1), lambda i: (0, 0)),
        ],
        out_specs=pl.BlockSpec((_BLK, 1), lambda i: (i, 0)),
        out_shape=jax.ShapeDtypeStruct((rows, 1), jnp.float32),
    )


def kernel(user_id, amount, merchant_id, merchant_city, merchant_state, mcc,
           zip_2, zip_4, user_avg_amount, merchant_avg_amount, card_id,
           use_chip, zip_1, errors, E_user_id, E_amount, E_mer_id, E_mer_ct,
           E_mer_st, E_mcc, E_zip2, E_zip4, E_user_avg, E_mer_avg, W_card,
           b_card, W_chip, b_chip, W_zip1, b_zip1, W_hidden, b_hidden, W_out,
           b_out):
    idx = jnp.pad(jnp.concatenate(
        [user_id, amount, merchant_id, merchant_city, zip_2,
         user_avg_amount, merchant_avg_amount],
        axis=1).astype(jnp.int32), ((0, 0), (0, 3))).T
    imst = merchant_state.astype(jnp.int32)
    imcc = mcc.astype(jnp.int32)
    iz4 = zip_4.astype(jnp.int32)

    tables = [E_user_id, E_amount, E_mer_id, E_mer_ct, E_zip2,
              E_user_avg, E_mer_avg]
    tables = [jnp.pad(t, ((0, 0), (0, _DP - _D))) for t in tables]
    tmst = jnp.pad(E_mer_st, ((0, _V2 - E_mer_st.shape[0]), (0, 0)))
    tmcc = jnp.pad(E_mcc, ((0, _V1 - E_mcc.shape[0]), (0, 0)))
    tz4 = jnp.pad(E_zip4, ((0, _V1 - E_zip4.shape[0]), (0, 0)))

    dpack = jnp.concatenate(
        [card_id, use_chip, zip_1, errors,
         jnp.zeros((_B, _DW - 17), jnp.float32)], axis=1)

    Wh = W_hidden
    segs = [
        Wh[0:64],
        Wh[128:192],
        Wh[200:264],
        Wh[264:328],
        Wh[648:712],
        Wh[776:840],
        Wh[840:904],
        jnp.zeros((_D, _H2), jnp.float32),
    ]
    wg = jnp.concatenate(segs, axis=0)
    wmst = Wh[328:392]
    wmcc = Wh[392:456] + Wh[456:520]
    wz4 = Wh[712:776]
    whd = jnp.concatenate([Wh[64:128], Wh[520:584], Wh[584:648]],
                          axis=0)
    we = jnp.zeros((_DW, _H2), jnp.float32).at[9:17].set(Wh[192:200])
    wd = (jnp.zeros((_DW, _H1), jnp.float32)
          .at[0:4, 0:64].set(W_card)
          .at[4:7, 64:128].set(W_chip)
          .at[7:9, 128:192].set(W_zip1))
    bd = jnp.concatenate([b_card, b_chip, b_zip1])[None, :]
    bh = b_hidden[None, :]
    bo = b_out[None, :]

    sc = _get_sc_gather(_S)
    mlp = _get_mlp(_S)
    outs = []
    for i in range(_NSPLIT):
        g = sc(*tables, lax.slice(idx, (0, i * _S), (10, (i + 1) * _S)))
        outs.append(mlp(g, lax.slice(dpack, (i * _S, 0), ((i + 1) * _S, _DW)),
                        lax.slice(imst, (i * _S, 0), ((i + 1) * _S, 1)),
                        lax.slice(imcc, (i * _S, 0), ((i + 1) * _S, 1)),
                        lax.slice(iz4, (i * _S, 0), ((i + 1) * _S, 1)),
                        tmst, tmcc, tz4, wmst, wmcc, wz4,
                        wg, wd, bd, whd, we, bh, W_out, bo))
    return jnp.concatenate(outs, axis=0)
